# baseline (speedup 1.0000x reference)
import functools

import jax
import jax.numpy as jnp
from jax import lax
from jax.experimental import pallas as pl
from jax.experimental.pallas import tpu as pltpu
from jax.experimental.pallas import tpu_sc as plsc

NC = 2
NS = 16
NW = NC * NS
NBUF = 8


@functools.partial(jax.jit, static_argnames=("b", "s", "d", "n_per_w"))
def _emb_lookup(emb_weight, x, b, s, d, n_per_w):
    mesh = plsc.VectorSubcoreMesh(core_axis_name="c", subcore_axis_name="s")
    n_groups = n_per_w // NBUF

    @functools.partial(
        pl.kernel,
        mesh=mesh,
        compiler_params=pltpu.CompilerParams(use_tc_tiling_on_sc=False),
        out_type=jax.ShapeDtypeStruct((b, s, d), jnp.float32),
        scratch_types=(
            [pltpu.VMEM((n_per_w, s), jnp.int32)]
            + [pltpu.VMEM((s, d), jnp.float32) for _ in range(NBUF)]
            + [pltpu.SemaphoreType.DMA for _ in range(2 * NBUF)]
        ),
    )
    def k(table_hbm, idx_hbm, out_hbm, idx_v, *bufs):
        rows = bufs[:NBUF]
        gsem = bufs[NBUF:2 * NBUF]
        osem = bufs[2 * NBUF:]
        wid = lax.axis_index("s") * NC + lax.axis_index("c")
        b0 = wid * n_per_w
        pltpu.sync_copy(idx_hbm.at[pl.ds(b0, n_per_w)], idx_v)

        def gather_start(j, bi):
            pltpu.async_copy(table_hbm.at[idx_v.at[j]], rows[bi], gsem[bi])

        def gather_wait(bi):
            pltpu.make_async_copy(
                table_hbm.at[idx_v.at[0]], rows[bi], gsem[bi]).wait()

        def out_start(j, bi):
            pltpu.async_copy(rows[bi], out_hbm.at[b0 + j], osem[bi])

        def out_wait(bi):
            pltpu.make_async_copy(rows[bi], out_hbm.at[b0], osem[bi]).wait()

        for bi in range(NBUF):
            gather_start(bi, bi)

        def outer(t, carry):
            base_j = t * NBUF
            for bi in range(NBUF):
                gather_wait(bi)
                out_start(base_j + bi, bi)
            for bi in range(NBUF):
                out_wait(bi)
                gather_start(base_j + NBUF + bi, bi)
            return carry

        lax.fori_loop(0, n_groups - 1, outer, 0)

        base_j = (n_groups - 1) * NBUF
        for bi in range(NBUF):
            gather_wait(bi)
            out_start(base_j + bi, bi)
        for bi in range(NBUF):
            out_wait(bi)

    return k(emb_weight, x)


def kernel(x, emb_weight):
    b, s = x.shape
    v, d = emb_weight.shape
    return _emb_lookup(emb_weight, x.astype(jnp.int32), b, s, d, b // NW)
